# baseline (speedup 1.0000x reference)
import functools

import jax
import jax.numpy as jnp
from jax import lax
from jax.experimental import pallas as pl
from jax.experimental.pallas import tpu as pltpu
from jax.experimental.pallas import tpu_sc as plsc

_NUM_CORES = 2
_NUM_SUBCORES = 16
_NW = _NUM_CORES * _NUM_SUBCORES


def _make_gather(N, D, C):
    per_w = N // _NW
    n_chunks = per_w // C
    mesh = plsc.VectorSubcoreMesh(core_axis_name="c", subcore_axis_name="s")

    @functools.partial(
        pl.kernel,
        out_type=jax.ShapeDtypeStruct((N, D), jnp.float32),
        mesh=mesh,
        scratch_types=[
            pltpu.VMEM((per_w,), jnp.int32),
            pltpu.VMEM((C, D), jnp.float32),
            pltpu.SemaphoreType.DMA,
        ],
    )
    def gather(table_hbm, ids_hbm, out_hbm, idx_v, buf, sem):
        wid = lax.axis_index("s") * _NUM_CORES + lax.axis_index("c")
        base = wid * per_w
        pltpu.sync_copy(ids_hbm.at[pl.ds(base, per_w)], idx_v)

        def body(j, carry):
            row0 = j * C
            pltpu.async_copy(
                table_hbm.at[idx_v.at[pl.ds(row0, C)]], buf, sem
            ).wait()
            pltpu.sync_copy(buf, out_hbm.at[pl.ds(base + row0, C)])
            return carry

        lax.fori_loop(0, n_chunks, body, 0)

    return gather


def _proj_kernel(bb, ll, x_ref, w_ref, b_ref, o_ref):
    x = x_ref[...]
    y = (
        jnp.dot(x, w_ref[...], preferred_element_type=jnp.float32)
        + b_ref[...]
    )
    o_ref[...] = y.reshape(bb, ll, -1)


def _proj(x_pad, w_pad, proj_b, B, L, BB):
    D = x_pad.shape[1]
    VO = w_pad.shape[1]
    grid = B // BB
    return pl.pallas_call(
        functools.partial(_proj_kernel, BB, L),
        grid=(grid,),
        in_specs=[
            pl.BlockSpec((BB * L, D), lambda i: (i, 0)),
            pl.BlockSpec((D, VO), lambda i: (0, 0)),
            pl.BlockSpec((1, VO), lambda i: (0, 0)),
        ],
        out_specs=pl.BlockSpec((BB, L, VO), lambda i: (i, 0, 0)),
        out_shape=jax.ShapeDtypeStruct((B, L, VO), jnp.float32),
    )(x_pad, w_pad, proj_b.reshape(1, VO))


def kernel(input_ids, embed_table, proj_w, proj_b):
    B, L = input_ids.shape
    V, E = embed_table.shape
    VO = proj_w.shape[1]
    N = B * L
    D = 128

    table_pad = jnp.pad(embed_table, ((0, 0), (0, D - E)))
    w_pad = jnp.pad(proj_w, ((0, D - E), (0, 0)))
    ids = input_ids.reshape(N).astype(jnp.int32)

    x_pad = _make_gather(N, D, C=80)(table_pad, ids)
    return _proj(x_pad, w_pad, proj_b, B, L, BB=16)


# speedup vs baseline: 1.0786x; 1.0786x over previous
import functools

import jax
import jax.numpy as jnp
from jax import lax
from jax.experimental import pallas as pl
from jax.experimental.pallas import tpu as pltpu


def _fused_kernel(rows, bb, ll, ids_ref, t_ref, w_ref, b_ref, o_ref, x_ref):
    def body(i, carry):
        idv = ids_ref[0, 0, i]
        x_ref[pl.ds(i, 1), :] = t_ref[pl.ds(idv, 1), :]
        return carry

    lax.fori_loop(0, rows, body, 0, unroll=8)
    y = (
        jnp.dot(x_ref[...], w_ref[...], preferred_element_type=jnp.float32)
        + b_ref[...]
    )
    o_ref[...] = y.reshape(bb, ll, -1)


def kernel(input_ids, embed_table, proj_w, proj_b):
    B, L = input_ids.shape
    V, E = embed_table.shape
    VO = proj_w.shape[1]
    N = B * L
    D = 128
    BB = 16
    ROWS = BB * L

    table_pad = jnp.pad(embed_table, ((0, 0), (0, D - E)))
    w_pad = jnp.pad(proj_w, ((0, D - E), (0, 0)))
    ids = input_ids.reshape(B // BB, 1, ROWS).astype(jnp.int32)

    return pl.pallas_call(
        functools.partial(_fused_kernel, ROWS, BB, L),
        grid=(B // BB,),
        in_specs=[
            pl.BlockSpec((1, 1, ROWS), lambda i: (i, 0, 0),
                         memory_space=pltpu.SMEM),
            pl.BlockSpec((V, D), lambda i: (0, 0)),
            pl.BlockSpec((D, VO), lambda i: (0, 0)),
            pl.BlockSpec((1, VO), lambda i: (0, 0)),
        ],
        out_specs=pl.BlockSpec((BB, L, VO), lambda i: (i, 0, 0)),
        out_shape=jax.ShapeDtypeStruct((B, L, VO), jnp.float32),
        scratch_shapes=[pltpu.VMEM((ROWS, D), jnp.float32)],
    )(ids, table_pad, w_pad, proj_b.reshape(1, VO))
